# Initial kernel scaffold; baseline (speedup 1.0000x reference)
import jax
import jax.numpy as jnp
from jax.experimental import pallas as pl


def kernel(x, edge_index, params):
    raise NotImplementedError("write your pallas kernel here")


# speedup vs baseline: 3.8881x; 3.8881x over previous
import functools

import jax
import jax.numpy as jnp
from jax import lax
from jax.experimental import pallas as pl
from jax.experimental.pallas import tpu as pltpu
from jax.experimental.pallas import tpu_sc as plsc

N = 10000
E = 320000
NUM_FEATURE = 128
HIDDIM = 128
NUM_LABEL = 40
HEADS = 4
C = HIDDIM // HEADS

NP = 10240
ROWBLK = 1024
NW = 32
EPW = E // NW
EB = 80
NCH = EPW // EB
NPW = NP // 16


def _mm_kernel(x_ref, w_ref, b_ref, o_ref, *, act):
    y = jnp.dot(x_ref[...], w_ref[...], preferred_element_type=jnp.float32)
    y = y + b_ref[...]
    if act == "relu":
        y = jnp.maximum(y, 0.0)
    o_ref[...] = y


def _dense(x, W, b, act=None):
    K = x.shape[1]
    M = W.shape[1]
    return pl.pallas_call(
        functools.partial(_mm_kernel, act=act),
        grid=(NP // ROWBLK,),
        in_specs=[
            pl.BlockSpec((ROWBLK, K), lambda i: (i, 0)),
            pl.BlockSpec((K, M), lambda i: (0, 0)),
            pl.BlockSpec((1, M), lambda i: (0, 0)),
        ],
        out_specs=pl.BlockSpec((ROWBLK, M), lambda i: (i, 0)),
        out_shape=jax.ShapeDtypeStruct((NP, M), jnp.float32),
    )(x, W, b.reshape(1, M))


def _gatprep_kernel(x_ref, w_ref, av_ref, o_h, o_a, o_es):
    h = jnp.dot(x_ref[...], w_ref[...], preferred_element_type=jnp.float32)
    o_h[...] = h
    hr = h.reshape(h.shape[0], HEADS, C)
    a_s = jnp.sum(hr * av_ref[0].reshape(1, HEADS, C), axis=-1)
    a_d = jnp.sum(hr * av_ref[1].reshape(1, HEADS, C), axis=-1)
    pad = jnp.zeros((h.shape[0], 12), jnp.float32)
    o_a[...] = jnp.stack(
        [jnp.concatenate([a_s, pad], axis=1),
         jnp.concatenate([a_d, pad], axis=1)], axis=0)
    sl = a_s + a_d
    sl = jnp.where(sl >= 0, sl, 0.2 * sl)
    es = jnp.exp(sl)
    esb = jnp.broadcast_to(es.reshape(h.shape[0], HEADS, 1),
                           (h.shape[0], HEADS, C)).reshape(h.shape[0], HIDDIM)
    o_es[...] = esb


def _gatprep(x, W, asrc, adst):
    av = jnp.stack([asrc.reshape(HIDDIM), adst.reshape(HIDDIM)])
    return pl.pallas_call(
        _gatprep_kernel,
        grid=(NP // ROWBLK,),
        in_specs=[
            pl.BlockSpec((ROWBLK, HIDDIM), lambda i: (i, 0)),
            pl.BlockSpec((HIDDIM, HIDDIM), lambda i: (0, 0)),
            pl.BlockSpec((2, HIDDIM), lambda i: (0, 0)),
        ],
        out_specs=[
            pl.BlockSpec((ROWBLK, HIDDIM), lambda i: (i, 0)),
            pl.BlockSpec((2, ROWBLK, 16), lambda i: (0, i, 0)),
            pl.BlockSpec((ROWBLK, HIDDIM), lambda i: (i, 0)),
        ],
        out_shape=[
            jax.ShapeDtypeStruct((NP, HIDDIM), jnp.float32),
            jax.ShapeDtypeStruct((2, NP, 16), jnp.float32),
            jax.ShapeDtypeStruct((NP, HIDDIM), jnp.float32),
        ],
    )(x, W, av)


def _gatcomb_kernel(n0_ref, n1_ref, d0_ref, d1_ref, h_ref, es_ref, b_ref, o_ref):
    num = n0_ref[0] + n1_ref[0] + es_ref[...] * h_ref[...]
    d16 = d0_ref[0] + d1_ref[0]
    d4 = d16[:, :HEADS]
    den = jnp.broadcast_to(d4.reshape(-1, HEADS, 1),
                           (d4.shape[0], HEADS, C)).reshape(-1, HIDDIM)
    den = den + es_ref[...]
    y = num / (den + 1e-16) + b_ref[...]
    o_ref[...] = jnp.maximum(y, 0.0)


def _gatcomb(num2, den2, h, es, b):
    blk = lambda i: (i, 0)
    return pl.pallas_call(
        _gatcomb_kernel,
        grid=(NP // ROWBLK,),
        in_specs=[
            pl.BlockSpec((1, ROWBLK, HIDDIM), lambda i: (0, i, 0)),
            pl.BlockSpec((1, ROWBLK, HIDDIM), lambda i: (1, i, 0)),
            pl.BlockSpec((1, ROWBLK, 16), lambda i: (0, i, 0)),
            pl.BlockSpec((1, ROWBLK, 16), lambda i: (1, i, 0)),
            pl.BlockSpec((ROWBLK, HIDDIM), blk),
            pl.BlockSpec((ROWBLK, HIDDIM), blk),
            pl.BlockSpec((1, HIDDIM), lambda i: (0, 0)),
        ],
        out_specs=pl.BlockSpec((ROWBLK, HIDDIM), blk),
        out_shape=jax.ShapeDtypeStruct((NP, HIDDIM), jnp.float32),
    )(num2, num2, den2, den2, h, es, b.reshape(1, HIDDIM))


def _gat_pass_body(h_hbm, a2_hbm, src_hbm, dst_hbm, z128_hbm, z16_hbm,
                   num_out, den_out,
                   sh_num, sh_den, srcv, dstv, hbuf, asb, adb, wbuf,
                   sem0, sem1, sem2):
    c = lax.axis_index("c")
    s = lax.axis_index("s")
    wid = c * 16 + s

    pltpu.sync_copy(z128_hbm, sh_num.at[pl.ds(s * NPW, NPW)])
    pltpu.sync_copy(z16_hbm, sh_den.at[pl.ds(s * NPW, NPW)])

    def _zw(e, carry):
        wbuf[e, :] = jnp.zeros((16,), jnp.float32)
        return carry
    lax.fori_loop(0, EB, _zw, 0)

    plsc.subcore_barrier()

    lane = lax.iota(jnp.int32, 16)
    rowoff = lane // 4
    coloff = lane % 4

    def _chunk(i, carry):
        base = wid * EPW + i * EB
        pltpu.sync_copy(src_hbm.at[pl.ds(base, EB)], srcv)
        pltpu.sync_copy(dst_hbm.at[pl.ds(base, EB)], dstv)
        cp0 = pltpu.async_copy(h_hbm.at[srcv], hbuf, sem0)
        cp1 = pltpu.async_copy(a2_hbm.at[0].at[srcv], asb, sem1)
        cp2 = pltpu.async_copy(a2_hbm.at[1].at[dstv], adb, sem2)
        cp0.wait()
        cp1.wait()
        cp2.wait()

        def _wg(g, carry2):
            rows = g * 4 + rowoff
            av = plsc.load_gather(asb, [rows, coloff])
            bv = plsc.load_gather(adb, [rows, coloff])
            ev = av + bv
            ev = jnp.where(ev >= 0, ev, 0.2 * ev)
            plsc.store_scatter(wbuf, [rows, coloff], jnp.exp(ev))
            return carry2
        lax.fori_loop(0, EB // 4, _wg, 0)

        def _se(e, carry2):
            er = jnp.full((16,), e, jnp.int32)
            for head in range(HEADS):
                wv = plsc.load_gather(wbuf, [er, jnp.full((16,), head, jnp.int32)])
                for r2 in range(2):
                    r = head * 2 + r2
                    hbuf[e, pl.ds(r * 16, 16)] = hbuf[e, pl.ds(r * 16, 16)] * wv
            return carry2
        lax.fori_loop(0, EB, _se, 0)

        pltpu.sync_copy(hbuf, sh_num.at[dstv], add=True)
        pltpu.sync_copy(wbuf, sh_den.at[dstv], add=True)
        return carry
    lax.fori_loop(0, NCH, _chunk, 0)

    plsc.subcore_barrier()

    pltpu.sync_copy(sh_num.at[pl.ds(s * NPW, NPW)],
                    num_out.at[c].at[pl.ds(s * NPW, NPW)])
    pltpu.sync_copy(sh_den.at[pl.ds(s * NPW, NPW)],
                    den_out.at[c].at[pl.ds(s * NPW, NPW)])


def _gat_pass(h, a2, srci, dsti, z128, z16):
    mesh = plsc.VectorSubcoreMesh(core_axis_name="c", subcore_axis_name="s")
    f = pl.kernel(
        _gat_pass_body,
        out_type=[
            jax.ShapeDtypeStruct((2, NP, HIDDIM), jnp.float32),
            jax.ShapeDtypeStruct((2, NP, 16), jnp.float32),
        ],
        mesh=mesh,
        compiler_params=pltpu.CompilerParams(needs_layout_passes=False,
                                             use_tc_tiling_on_sc=False),
        scratch_types=[
            pltpu.VMEM_SHARED((NP, HIDDIM), jnp.float32),
            pltpu.VMEM_SHARED((NP, 16), jnp.float32),
            pltpu.VMEM((EB,), jnp.int32),
            pltpu.VMEM((EB,), jnp.int32),
            pltpu.VMEM((EB, HIDDIM), jnp.float32),
            pltpu.VMEM((EB, 16), jnp.float32),
            pltpu.VMEM((EB, 16), jnp.float32),
            pltpu.VMEM((EB, 16), jnp.float32),
            pltpu.SemaphoreType.DMA,
            pltpu.SemaphoreType.DMA,
            pltpu.SemaphoreType.DMA,
        ],
    )
    return f(h, a2, srci, dsti, z128, z16)


def _gat_sc(h, a2, es, srci, dsti, b, z128, z16):
    num2, den2 = _gat_pass(h, a2, srci, dsti, z128, z16)
    return _gatcomb(num2, den2, h, es, b)


def _seg_softmax(logits, seg, n):
    m = jax.ops.segment_max(logits, seg, num_segments=n)
    m = jnp.where(jnp.isfinite(m), m, 0.0)
    e = jnp.exp(logits - m[seg])
    d = jax.ops.segment_sum(e, seg, num_segments=n)
    return e / (d[seg] + 1e-16)


def _trans(q, k, v, skip, src, dst, n):
    qq = q[:n].reshape(n, HEADS, C)
    kk = k[:n].reshape(n, HEADS, C)
    vv = v[:n].reshape(n, HEADS, C)
    e = jnp.sum(qq[dst] * kk[src], axis=-1) / (float(C) ** 0.5)
    a = _seg_softmax(e, dst, n)
    out = jax.ops.segment_sum(vv[src] * a[:, :, None], dst, num_segments=n)
    return out.reshape(n, HEADS * C) + skip[:n]


def _context(xp, src, dst, srci, dsti, p, l, z128, z16):
    x1 = _dense(xp, p['W_fc%d0' % l], p['b_fc%d0' % l], "relu")
    x2 = _dense(xp, p['W_fc%d1' % l], p['b_fc%d1' % l], "relu")
    x3 = _dense(xp, p['W_fc%d2' % l], p['b_fc%d2' % l], "relu")
    x4 = _dense(xp, p['W_fc%d3' % l], p['b_fc%d3' % l], "relu")
    g = []
    for j, xj in ((1, x1), (2, x2), (3, x3)):
        h, a2, es = _gatprep(xj, p['gat%d%d_W' % (l, j)],
                             p['gat%d%d_asrc' % (l, j)], p['gat%d%d_adst' % (l, j)])
        g.append(_gat_sc(h, a2, es, srci, dsti, p['gat%d%d_b' % (l, j)], z128, z16))
    oc = g[0] * g[1]
    q = _dense(oc, p['trans%d_Wq' % l], p['trans%d_bq' % l])
    k = _dense(oc, p['trans%d_Wk' % l], p['trans%d_bk' % l])
    v = _dense(oc, p['trans%d_Wv' % l], p['trans%d_bv' % l])
    skip = _dense(oc, p['trans%d_Wskip' % l], p['trans%d_bskip' % l])
    m = jax.nn.relu(_trans(q, k, v, skip, src, dst, N))
    m = jax.nn.softmax(m, axis=1)
    z = m * g[2][:N]
    x4n = x4[:N]
    cat = jnp.concatenate([z, x4n, z - x4n], axis=1)
    G = jax.nn.sigmoid(cat @ p['W_g%d' % l] + p['b_g%d' % l])
    Y = G * z + (1.0 - G) * x4n
    return jax.nn.leaky_relu(Y, 0.25)


def kernel(x, edge_index, params):
    src = edge_index[0]
    dst = edge_index[1]
    srci = edge_index[0].astype(jnp.int32)
    dsti = edge_index[1].astype(jnp.int32)
    z128 = jnp.zeros((NPW, HIDDIM), jnp.float32)
    z16 = jnp.zeros((NPW, 16), jnp.float32)
    xp = jnp.zeros((NP, NUM_FEATURE), jnp.float32).at[:N].set(x)
    K0 = _context(xp, src, dst, srci, dsti, params, 0, z128, z16)
    Kp = jnp.zeros((NP, HIDDIM), jnp.float32).at[:N].set(K0)
    K1 = _dense(Kp, params['WK'], params['bK'])
    K1 = K1.at[N:].set(0.0)
    K2 = _context(K1, src, dst, srci, dsti, params, 1, z128, z16)
    Kp2 = jnp.zeros((NP, HIDDIM), jnp.float32).at[:N].set(K2)
    out = _dense(Kp2, params['Wout'], params['bout'])[:N]
    return jax.nn.log_softmax(out, axis=1)
